# baseline (speedup 1.0000x reference)
import functools

import jax
import jax.numpy as jnp
from jax import lax
from jax.experimental import pallas as pl
from jax.experimental.pallas import tpu as pltpu
from jax.experimental.pallas import tpu_sc as plsc

_E = 8
_K = 2
_BM = 128
_NB = 39
_SPAD = _NB * _BM
_NC = 2
_NS = 16
_NW = _NC * _NS
_L = 16


def _routing_body(logits_ref, p0_ref, p1_ref, w0_ref, w1_ref, meta_ref):
    lg = logits_ref[...]
    T = lg.shape[0]
    m = jnp.max(lg, axis=-1, keepdims=True)
    ex = jnp.exp(lg - m)
    p = ex / jnp.sum(ex, axis=-1, keepdims=True)
    lane = lax.broadcasted_iota(jnp.int32, p.shape, 1)
    m1 = jnp.max(p, axis=-1, keepdims=True)
    l1 = jnp.min(jnp.where(p >= m1, lane, _E), axis=-1, keepdims=True)
    mask1 = lane == l1
    pm = jnp.where(mask1, -1.0, p)
    m2 = jnp.max(pm, axis=-1, keepdims=True)
    l2 = jnp.min(jnp.where(pm >= m2, lane, _E), axis=-1, keepdims=True)
    mask2 = lane == l2
    denom = m1 + m2
    w0_ref[...] = jnp.broadcast_to(m1 / denom, (T, _L))
    w1_ref[...] = jnp.broadcast_to(m2 / denom, (T, _L))

    a1 = mask1.astype(jnp.float32)
    a2 = mask2.astype(jnp.float32)
    A = a1 + a2
    CH = 512
    ri = lax.broadcasted_iota(jnp.int32, (CH, CH), 0)
    ci = lax.broadcasted_iota(jnp.int32, (CH, CH), 1)
    tril = (ci < ri).astype(jnp.bfloat16)
    carry = jnp.zeros((1, _E), jnp.float32)
    chunks = []
    for b in range(T // CH):
        Ab = A[b * CH:(b + 1) * CH, :]
        Rb = jnp.dot(tril, Ab.astype(jnp.bfloat16),
                     preferred_element_type=jnp.float32) + carry
        chunks.append(Rb)
        carry = carry + jnp.sum(Ab, axis=0, keepdims=True)
    Rex = jnp.concatenate(chunks, axis=0)
    counts = carry
    nblk = jnp.floor((counts + (_BM - 1)) * (1.0 / _BM))
    r8 = lax.broadcasted_iota(jnp.int32, (_E, _E), 0)
    c8 = lax.broadcasted_iota(jnp.int32, (_E, _E), 1)
    tril8 = (r8 < c8).astype(jnp.float32)
    blkoff = jnp.dot(nblk, tril8, preferred_element_type=jnp.float32)
    poff = blkoff * float(_BM)
    base = poff + Rex
    p0_ref[...] = jnp.sum(jnp.where(mask1, base, 0.0), axis=-1,
                          keepdims=True).astype(jnp.int32)
    p1_ref[...] = jnp.sum(jnp.where(mask2, base, 0.0), axis=-1,
                          keepdims=True).astype(jnp.int32)

    cumblk = blkoff + nblk
    bi = lax.broadcasted_iota(jnp.int32, (128, _E), 0).astype(jnp.float32)
    be = jnp.sum((bi >= cumblk).astype(jnp.float32), axis=-1, keepdims=True)
    meta_ref[...] = jnp.minimum(be, float(_E - 1)).astype(jnp.int32)


def _dispatch_body(p0_hbm, p1_hbm, x_hbm, xs_hbm,
                   i0a, i0b, i1a, i1b, rows0, rows1,
                   semg0, semg1, sems0, sems1):
    wid = lax.axis_index("s") * _NC + lax.axis_index("c")
    tbase = wid * 64
    idx0 = [i0a, i0b]
    idx1 = [i1a, i1b]
    rows = [rows0, rows1]
    semg = [semg0, semg1]
    for c in range(2):
        pltpu.sync_copy(p0_hbm.at[pl.ds(tbase + c * 32, 32)], idx0[c])
        pltpu.sync_copy(p1_hbm.at[pl.ds(tbase + c * 32, 32)], idx1[c])
    gets = [
        pltpu.async_copy(x_hbm.at[pl.ds(tbase, 32)], rows0, semg0),
        pltpu.async_copy(x_hbm.at[pl.ds(tbase + 32, 32)], rows1, semg1),
    ]
    for c in range(2):
        gets[c].wait()
        put0 = pltpu.async_copy(rows[c], xs_hbm.at[idx0[c]], sems0)
        put1 = pltpu.async_copy(rows[c], xs_hbm.at[idx1[c]], sems1)
        put0.wait()
        put1.wait()


def _gmm_body(be_ref, xs_ref, wg_ref, wu_ref, wd_ref, ys_ref):
    xb = xs_ref[...].astype(jnp.bfloat16)
    wg = wg_ref[0].astype(jnp.bfloat16)
    wu = wu_ref[0].astype(jnp.bfloat16)
    wd = wd_ref[0].astype(jnp.bfloat16)
    gate = jnp.dot(xb, wg, preferred_element_type=jnp.float32)
    up = jnp.dot(xb, wu, preferred_element_type=jnp.float32)
    h = (gate * jax.nn.sigmoid(gate) * up).astype(jnp.bfloat16)
    ys_ref[...] = jnp.dot(h, wd, preferred_element_type=jnp.float32)


def _combine_body(p0_hbm, p1_hbm, w0_hbm, w1_hbm, ys_hbm, out_hbm,
                  i0a, i0b, i0c, i0d, i1a, i1b, i1c, i1d, w0v, w1v,
                  r0a, r0b, r1a, r1b, outv0, outv1,
                  sg0a, sg0b, sg1a, sg1b, semo0, semo1):
    H = r0a.shape[-1]
    wid = lax.axis_index("s") * _NC + lax.axis_index("c")
    tbase = wid * 64
    idx0 = [i0a, i0b, i0c, i0d]
    idx1 = [i1a, i1b, i1c, i1d]
    r0 = [r0a, r0b]
    r1 = [r1a, r1b]
    outs = [outv0, outv1]
    sg0 = [sg0a, sg0b]
    sg1 = [sg1a, sg1b]
    semo = [semo0, semo1]
    for c in range(4):
        pltpu.sync_copy(p0_hbm.at[pl.ds(tbase + c * 16, 16)], idx0[c])
        pltpu.sync_copy(p1_hbm.at[pl.ds(tbase + c * 16, 16)], idx1[c])
    pltpu.sync_copy(w0_hbm.at[pl.ds(tbase, 64)], w0v)
    pltpu.sync_copy(w1_hbm.at[pl.ds(tbase, 64)], w1v)
    g0 = [None] * 4
    g1 = [None] * 4
    puts = [None] * 4
    for c in range(2):
        g0[c] = pltpu.async_copy(ys_hbm.at[idx0[c]], r0[c], sg0[c])
        g1[c] = pltpu.async_copy(ys_hbm.at[idx1[c]], r1[c], sg1[c])
    for c in range(4):
        g0[c].wait()
        g1[c].wait()
        if c >= 2:
            puts[c - 2].wait()
        ra = r0[c % 2]
        rb = r1[c % 2]
        ov = outs[c % 2]
        w0s = [w0v[c * 16 + t, :] for t in range(16)]
        w1s = [w1v[c * 16 + t, :] for t in range(16)]

        def body(v, _, ra=ra, rb=rb, ov=ov, w0s=w0s, w1s=w1s):
            sl = pl.ds(v * _L, _L)
            for t in range(16):
                ov[t, sl] = w0s[t] * ra[t, sl] + w1s[t] * rb[t, sl]
            return 0

        lax.fori_loop(0, H // _L, body, 0)
        if c + 2 < 4:
            g0[c + 2] = pltpu.async_copy(
                ys_hbm.at[idx0[c + 2]], r0[c % 2], sg0[c % 2])
            g1[c + 2] = pltpu.async_copy(
                ys_hbm.at[idx1[c + 2]], r1[c % 2], sg1[c % 2])
        puts[c] = pltpu.async_copy(
            ov, out_hbm.at[pl.ds(tbase + c * 16, 16)], semo[c % 2])
    puts[2].wait()
    puts[3].wait()


def kernel(x, router_logits, w_gate, w_up, w_down):
    T, H = x.shape
    E, _, F = w_gate.shape

    pos0, pos1, w0r, w1r, meta = pl.pallas_call(
        _routing_body,
        out_shape=(
            jax.ShapeDtypeStruct((T, 1), jnp.int32),
            jax.ShapeDtypeStruct((T, 1), jnp.int32),
            jax.ShapeDtypeStruct((T, _L), jnp.float32),
            jax.ShapeDtypeStruct((T, _L), jnp.float32),
            jax.ShapeDtypeStruct((128, 1), jnp.int32),
        ),
    )(router_logits)

    p0f = pos0.reshape(T)
    p1f = pos1.reshape(T)

    mesh = plsc.VectorSubcoreMesh(core_axis_name="c", subcore_axis_name="s")

    dispatch = functools.partial(
        pl.kernel,
        out_type=jax.ShapeDtypeStruct((_SPAD, H), jnp.float32),
        mesh=mesh,
        scratch_types=(
            [pltpu.VMEM((32,), jnp.int32)] * 4
            + [pltpu.VMEM((32, H), jnp.float32)] * 2
            + [pltpu.SemaphoreType.DMA] * 4
        ),
    )(_dispatch_body)
    xs = dispatch(p0f, p1f, x)

    ys = pl.pallas_call(
        _gmm_body,
        grid_spec=pltpu.PrefetchScalarGridSpec(
            num_scalar_prefetch=1,
            grid=(_NB,),
            in_specs=[
                pl.BlockSpec((_BM, H), lambda i, be: (i, 0)),
                pl.BlockSpec((1, H, F), lambda i, be: (be[i, 0], 0, 0)),
                pl.BlockSpec((1, H, F), lambda i, be: (be[i, 0], 0, 0)),
                pl.BlockSpec((1, F, H), lambda i, be: (be[i, 0], 0, 0)),
            ],
            out_specs=pl.BlockSpec((_BM, H), lambda i, be: (i, 0)),
        ),
        out_shape=jax.ShapeDtypeStruct((_SPAD, H), jnp.float32),
    )(meta, xs, w_gate, w_up, w_down)

    combine = functools.partial(
        pl.kernel,
        out_type=jax.ShapeDtypeStruct((T, H), jnp.float32),
        mesh=mesh,
        scratch_types=(
            [pltpu.VMEM((16,), jnp.int32)] * 8
            + [pltpu.VMEM((64, _L), jnp.float32)] * 2
            + [pltpu.VMEM((16, H), jnp.float32)] * 4
            + [pltpu.VMEM((16, H), jnp.float32)] * 2
            + [pltpu.SemaphoreType.DMA] * 6
        ),
    )(_combine_body)
    out = combine(p0f, p1f, w0r, w1r, ys)
    return out


# speedup vs baseline: 1.0423x; 1.0423x over previous
import functools

import jax
import jax.numpy as jnp
from jax import lax
from jax.experimental import pallas as pl
from jax.experimental.pallas import tpu as pltpu
from jax.experimental.pallas import tpu_sc as plsc

_E = 8
_K = 2
_BM = 256
_NB = 23
_SPAD = _NB * _BM
_NC = 2
_NS = 16
_NW = _NC * _NS
_L = 16


def _routing_body(logits_ref, p0_ref, p1_ref, w0_ref, w1_ref, meta_ref):
    lg = logits_ref[...]
    T = lg.shape[0]
    m = jnp.max(lg, axis=-1, keepdims=True)
    ex = jnp.exp(lg - m)
    p = ex / jnp.sum(ex, axis=-1, keepdims=True)
    lane = lax.broadcasted_iota(jnp.int32, p.shape, 1)
    m1 = jnp.max(p, axis=-1, keepdims=True)
    l1 = jnp.min(jnp.where(p >= m1, lane, _E), axis=-1, keepdims=True)
    mask1 = lane == l1
    pm = jnp.where(mask1, -1.0, p)
    m2 = jnp.max(pm, axis=-1, keepdims=True)
    l2 = jnp.min(jnp.where(pm >= m2, lane, _E), axis=-1, keepdims=True)
    mask2 = lane == l2
    denom = m1 + m2
    w0_ref[...] = jnp.broadcast_to(m1 / denom, (T, _L))
    w1_ref[...] = jnp.broadcast_to(m2 / denom, (T, _L))

    a1 = mask1.astype(jnp.float32)
    a2 = mask2.astype(jnp.float32)
    A = a1 + a2
    CH = 512
    ri = lax.broadcasted_iota(jnp.int32, (CH, CH), 0)
    ci = lax.broadcasted_iota(jnp.int32, (CH, CH), 1)
    tril = (ci < ri).astype(jnp.bfloat16)
    carry = jnp.zeros((1, _E), jnp.float32)
    chunks = []
    for b in range(T // CH):
        Ab = A[b * CH:(b + 1) * CH, :]
        Rb = jnp.dot(tril, Ab.astype(jnp.bfloat16),
                     preferred_element_type=jnp.float32) + carry
        chunks.append(Rb)
        carry = carry + jnp.sum(Ab, axis=0, keepdims=True)
    Rex = jnp.concatenate(chunks, axis=0)
    counts = carry
    nblk = jnp.floor((counts + (_BM - 1)) * (1.0 / _BM))
    r8 = lax.broadcasted_iota(jnp.int32, (_E, _E), 0)
    c8 = lax.broadcasted_iota(jnp.int32, (_E, _E), 1)
    tril8 = (r8 < c8).astype(jnp.float32)
    blkoff = jnp.dot(nblk, tril8, preferred_element_type=jnp.float32)
    poff = blkoff * float(_BM)
    base = poff + Rex
    p0_ref[...] = jnp.sum(jnp.where(mask1, base, 0.0), axis=-1,
                          keepdims=True).astype(jnp.int32).reshape(T)
    p1_ref[...] = jnp.sum(jnp.where(mask2, base, 0.0), axis=-1,
                          keepdims=True).astype(jnp.int32).reshape(T)

    cumblk = blkoff + nblk
    bi = lax.broadcasted_iota(jnp.int32, (128, _E), 0).astype(jnp.float32)
    be = jnp.sum((bi >= cumblk).astype(jnp.float32), axis=-1, keepdims=True)
    meta_ref[...] = jnp.minimum(be, float(_E - 1)).astype(jnp.int32)


def _dispatch_body(p0_hbm, p1_hbm, x_hbm, xs_hbm,
                   i0a, i0b, i1a, i1b, rows0, rows1,
                   semg0, semg1, sems0, sems1):
    wid = lax.axis_index("s") * _NC + lax.axis_index("c")
    tbase = wid * 64
    idx0 = [i0a, i0b]
    idx1 = [i1a, i1b]
    rows = [rows0, rows1]
    semg = [semg0, semg1]
    for c in range(2):
        pltpu.sync_copy(p0_hbm.at[pl.ds(tbase + c * 32, 32)], idx0[c])
        pltpu.sync_copy(p1_hbm.at[pl.ds(tbase + c * 32, 32)], idx1[c])
    gets = [
        pltpu.async_copy(x_hbm.at[pl.ds(tbase, 32)], rows0, semg0),
        pltpu.async_copy(x_hbm.at[pl.ds(tbase + 32, 32)], rows1, semg1),
    ]
    for c in range(2):
        gets[c].wait()
        put0 = pltpu.async_copy(rows[c], xs_hbm.at[idx0[c]], sems0)
        put1 = pltpu.async_copy(rows[c], xs_hbm.at[idx1[c]], sems1)
        put0.wait()
        put1.wait()


def _gmm_body(be_ref, xs_ref, wg_ref, wu_ref, wd_ref, ys_ref):
    xb = xs_ref[...].astype(jnp.bfloat16)
    wg = wg_ref[0].astype(jnp.bfloat16)
    wu = wu_ref[0].astype(jnp.bfloat16)
    wd = wd_ref[0].astype(jnp.bfloat16)
    gate = jnp.dot(xb, wg, preferred_element_type=jnp.float32)
    up = jnp.dot(xb, wu, preferred_element_type=jnp.float32)
    h = (gate * jax.nn.sigmoid(gate) * up).astype(jnp.bfloat16)
    ys_ref[...] = jnp.dot(h, wd, preferred_element_type=jnp.float32)


def _combine_body(p0_hbm, p1_hbm, w0_hbm, w1_hbm, ys_hbm, out_hbm,
                  i0a, i0b, i0c, i0d, i1a, i1b, i1c, i1d, w0v, w1v,
                  r0a, r0b, r1a, r1b, outv0, outv1,
                  sg0a, sg0b, sg1a, sg1b, semo0, semo1):
    H = r0a.shape[-1]
    wid = lax.axis_index("s") * _NC + lax.axis_index("c")
    tbase = wid * 64
    idx0 = [i0a, i0b, i0c, i0d]
    idx1 = [i1a, i1b, i1c, i1d]
    r0 = [r0a, r0b]
    r1 = [r1a, r1b]
    outs = [outv0, outv1]
    sg0 = [sg0a, sg0b]
    sg1 = [sg1a, sg1b]
    semo = [semo0, semo1]
    for c in range(4):
        pltpu.sync_copy(p0_hbm.at[pl.ds(tbase + c * 16, 16)], idx0[c])
        pltpu.sync_copy(p1_hbm.at[pl.ds(tbase + c * 16, 16)], idx1[c])
    pltpu.sync_copy(w0_hbm.at[pl.ds(tbase, 64)], w0v)
    pltpu.sync_copy(w1_hbm.at[pl.ds(tbase, 64)], w1v)
    g0 = [None] * 4
    g1 = [None] * 4
    puts = [None] * 4
    for c in range(2):
        g0[c] = pltpu.async_copy(ys_hbm.at[idx0[c]], r0[c], sg0[c])
        g1[c] = pltpu.async_copy(ys_hbm.at[idx1[c]], r1[c], sg1[c])
    for c in range(4):
        g0[c].wait()
        g1[c].wait()
        if c >= 2:
            puts[c - 2].wait()
        ra = r0[c % 2]
        rb = r1[c % 2]
        ov = outs[c % 2]
        w0s = [w0v[c * 16 + t, :] for t in range(16)]
        w1s = [w1v[c * 16 + t, :] for t in range(16)]

        def body(v, _, ra=ra, rb=rb, ov=ov, w0s=w0s, w1s=w1s):
            sl = pl.ds(v * _L, _L)
            for t in range(16):
                ov[t, sl] = w0s[t] * ra[t, sl] + w1s[t] * rb[t, sl]
            return 0

        lax.fori_loop(0, H // _L, body, 0)
        if c + 2 < 4:
            g0[c + 2] = pltpu.async_copy(
                ys_hbm.at[idx0[c + 2]], r0[c % 2], sg0[c % 2])
            g1[c + 2] = pltpu.async_copy(
                ys_hbm.at[idx1[c + 2]], r1[c % 2], sg1[c % 2])
        puts[c] = pltpu.async_copy(
            ov, out_hbm.at[pl.ds(tbase + c * 16, 16)], semo[c % 2])
    puts[2].wait()
    puts[3].wait()


def kernel(x, router_logits, w_gate, w_up, w_down):
    T, H = x.shape
    E, _, F = w_gate.shape

    pos0, pos1, w0r, w1r, meta = pl.pallas_call(
        _routing_body,
        out_shape=(
            jax.ShapeDtypeStruct((T,), jnp.int32),
            jax.ShapeDtypeStruct((T,), jnp.int32),
            jax.ShapeDtypeStruct((T, _L), jnp.float32),
            jax.ShapeDtypeStruct((T, _L), jnp.float32),
            jax.ShapeDtypeStruct((128, 1), jnp.int32),
        ),
    )(router_logits)

    p0f = pos0
    p1f = pos1

    mesh = plsc.VectorSubcoreMesh(core_axis_name="c", subcore_axis_name="s")

    dispatch = functools.partial(
        pl.kernel,
        out_type=jax.ShapeDtypeStruct((_SPAD, H), jnp.float32),
        mesh=mesh,
        scratch_types=(
            [pltpu.VMEM((32,), jnp.int32)] * 4
            + [pltpu.VMEM((32, H), jnp.float32)] * 2
            + [pltpu.SemaphoreType.DMA] * 4
        ),
    )(_dispatch_body)
    xs = dispatch(p0f, p1f, x)

    ys = pl.pallas_call(
        _gmm_body,
        grid_spec=pltpu.PrefetchScalarGridSpec(
            num_scalar_prefetch=1,
            grid=(_NB,),
            in_specs=[
                pl.BlockSpec((_BM, H), lambda i, be: (i, 0)),
                pl.BlockSpec((1, H, F), lambda i, be: (be[i, 0], 0, 0)),
                pl.BlockSpec((1, H, F), lambda i, be: (be[i, 0], 0, 0)),
                pl.BlockSpec((1, F, H), lambda i, be: (be[i, 0], 0, 0)),
            ],
            out_specs=pl.BlockSpec((_BM, H), lambda i, be: (i, 0)),
        ),
        out_shape=jax.ShapeDtypeStruct((_SPAD, H), jnp.float32),
    )(meta, xs, w_gate, w_up, w_down)

    combine = functools.partial(
        pl.kernel,
        out_type=jax.ShapeDtypeStruct((T, H), jnp.float32),
        mesh=mesh,
        scratch_types=(
            [pltpu.VMEM((16,), jnp.int32)] * 8
            + [pltpu.VMEM((64, _L), jnp.float32)] * 2
            + [pltpu.VMEM((16, H), jnp.float32)] * 4
            + [pltpu.VMEM((16, H), jnp.float32)] * 2
            + [pltpu.SemaphoreType.DMA] * 6
        ),
    )(_combine_body)
    out = combine(p0f, p1f, w0r, w1r, ys)
    return out


# speedup vs baseline: 1.0730x; 1.0295x over previous
import functools

import jax
import jax.numpy as jnp
from jax import lax
from jax.experimental import pallas as pl
from jax.experimental.pallas import tpu as pltpu
from jax.experimental.pallas import tpu_sc as plsc

_E = 8
_K = 2
_BM = 256
_NB = 23
_SPAD = _NB * _BM
_NC = 2
_NS = 16
_NW = _NC * _NS
_L = 16


def _routing_body(logits_ref, p0_ref, p1_ref, w0_ref, w1_ref, meta_ref):
    lg = logits_ref[...]
    T = lg.shape[0]
    m = jnp.max(lg, axis=-1, keepdims=True)
    ex = jnp.exp(lg - m)
    p = ex / jnp.sum(ex, axis=-1, keepdims=True)
    lane = lax.broadcasted_iota(jnp.int32, p.shape, 1)
    m1 = jnp.max(p, axis=-1, keepdims=True)
    l1 = jnp.min(jnp.where(p >= m1, lane, _E), axis=-1, keepdims=True)
    mask1 = lane == l1
    pm = jnp.where(mask1, -1.0, p)
    m2 = jnp.max(pm, axis=-1, keepdims=True)
    l2 = jnp.min(jnp.where(pm >= m2, lane, _E), axis=-1, keepdims=True)
    mask2 = lane == l2
    denom = m1 + m2
    w0_ref[...] = jnp.broadcast_to(m1 / denom, (T, _L))
    w1_ref[...] = jnp.broadcast_to(m2 / denom, (T, _L))

    a1 = mask1.astype(jnp.float32)
    a2 = mask2.astype(jnp.float32)
    A = a1 + a2
    CH = 512
    ri = lax.broadcasted_iota(jnp.int32, (CH, CH), 0)
    ci = lax.broadcasted_iota(jnp.int32, (CH, CH), 1)
    tril = (ci < ri).astype(jnp.bfloat16)
    carry = jnp.zeros((1, _E), jnp.float32)
    chunks = []
    for b in range(T // CH):
        Ab = A[b * CH:(b + 1) * CH, :]
        Rb = jnp.dot(tril, Ab.astype(jnp.bfloat16),
                     preferred_element_type=jnp.float32) + carry
        chunks.append(Rb)
        carry = carry + jnp.sum(Ab, axis=0, keepdims=True)
    Rex = jnp.concatenate(chunks, axis=0)
    counts = carry
    nblk = jnp.floor((counts + (_BM - 1)) * (1.0 / _BM))
    r8 = lax.broadcasted_iota(jnp.int32, (_E, _E), 0)
    c8 = lax.broadcasted_iota(jnp.int32, (_E, _E), 1)
    tril8 = (r8 < c8).astype(jnp.float32)
    blkoff = jnp.dot(nblk, tril8, preferred_element_type=jnp.float32)
    poff = blkoff * float(_BM)
    base = poff + Rex
    p0_ref[...] = jnp.sum(jnp.where(mask1, base, 0.0), axis=-1,
                          keepdims=True).astype(jnp.int32).reshape(T)
    p1_ref[...] = jnp.sum(jnp.where(mask2, base, 0.0), axis=-1,
                          keepdims=True).astype(jnp.int32).reshape(T)

    cumblk = blkoff + nblk
    bi = lax.broadcasted_iota(jnp.int32, (128, _E), 0).astype(jnp.float32)
    be = jnp.sum((bi >= cumblk).astype(jnp.float32), axis=-1, keepdims=True)
    meta_ref[...] = jnp.minimum(be, float(_E - 1)).astype(jnp.int32)


def _dispatch_body(p0_hbm, p1_hbm, x_hbm, xs_hbm,
                   i0a, i0b, i1a, i1b, rows0, rows1,
                   semg0, semg1, sems0, sems1, sems2, sems3):
    wid = lax.axis_index("s") * _NC + lax.axis_index("c")
    tbase = wid * 64
    idx0 = [i0a, i0b]
    idx1 = [i1a, i1b]
    rows = [rows0, rows1]
    gets = [
        pltpu.async_copy(x_hbm.at[pl.ds(tbase, 32)], rows0, semg0),
        pltpu.async_copy(x_hbm.at[pl.ds(tbase + 32, 32)], rows1, semg1),
    ]
    for c in range(2):
        pltpu.sync_copy(p0_hbm.at[pl.ds(tbase + c * 32, 32)], idx0[c])
        pltpu.sync_copy(p1_hbm.at[pl.ds(tbase + c * 32, 32)], idx1[c])
    puts = []
    sems = [sems0, sems1, sems2, sems3]
    for c in range(2):
        gets[c].wait()
        puts.append(pltpu.async_copy(rows[c], xs_hbm.at[idx0[c]], sems[2 * c]))
        puts.append(pltpu.async_copy(rows[c], xs_hbm.at[idx1[c]], sems[2 * c + 1]))
    for p in puts:
        p.wait()


def _gmm_body(be_ref, xs_ref, wg_ref, wu_ref, wd_ref, ys_ref):
    xb = xs_ref[...].astype(jnp.bfloat16)
    wg = wg_ref[0].astype(jnp.bfloat16)
    wu = wu_ref[0].astype(jnp.bfloat16)
    wd = wd_ref[0].astype(jnp.bfloat16)
    gate = jnp.dot(xb, wg, preferred_element_type=jnp.float32)
    up = jnp.dot(xb, wu, preferred_element_type=jnp.float32)
    h = (gate * jax.nn.sigmoid(gate) * up).astype(jnp.bfloat16)
    ys_ref[...] = jnp.dot(h, wd, preferred_element_type=jnp.float32)


def _combine_body(p0_hbm, p1_hbm, w0_hbm, w1_hbm, ys_hbm, out_hbm,
                  i0a, i0b, i0c, i0d, i1a, i1b, i1c, i1d, w0v, w1v,
                  r0a, r0b, r1a, r1b, outv0, outv1,
                  sg0a, sg0b, sg1a, sg1b, semo0, semo1):
    H = r0a.shape[-1]
    wid = lax.axis_index("s") * _NC + lax.axis_index("c")
    tbase = wid * 64
    idx0 = [i0a, i0b, i0c, i0d]
    idx1 = [i1a, i1b, i1c, i1d]
    r0 = [r0a, r0b]
    r1 = [r1a, r1b]
    outs = [outv0, outv1]
    sg0 = [sg0a, sg0b]
    sg1 = [sg1a, sg1b]
    semo = [semo0, semo1]
    g0 = [None] * 4
    g1 = [None] * 4
    puts = [None] * 4
    for c in range(2):
        pltpu.sync_copy(p0_hbm.at[pl.ds(tbase + c * 16, 16)], idx0[c])
        pltpu.sync_copy(p1_hbm.at[pl.ds(tbase + c * 16, 16)], idx1[c])
        g0[c] = pltpu.async_copy(ys_hbm.at[idx0[c]], r0[c], sg0[c])
        g1[c] = pltpu.async_copy(ys_hbm.at[idx1[c]], r1[c], sg1[c])
    for c in range(2, 4):
        pltpu.sync_copy(p0_hbm.at[pl.ds(tbase + c * 16, 16)], idx0[c])
        pltpu.sync_copy(p1_hbm.at[pl.ds(tbase + c * 16, 16)], idx1[c])
    pltpu.sync_copy(w0_hbm.at[pl.ds(tbase, 64)], w0v)
    pltpu.sync_copy(w1_hbm.at[pl.ds(tbase, 64)], w1v)
    for c in range(4):
        g0[c].wait()
        g1[c].wait()
        if c >= 2:
            puts[c - 2].wait()
        ra = r0[c % 2]
        rb = r1[c % 2]
        ov = outs[c % 2]
        w0s = [w0v[c * 16 + t, :] for t in range(16)]
        w1s = [w1v[c * 16 + t, :] for t in range(16)]

        def body(v, _, ra=ra, rb=rb, ov=ov, w0s=w0s, w1s=w1s):
            sl = pl.ds(v * _L, _L)
            for t in range(16):
                ov[t, sl] = w0s[t] * ra[t, sl] + w1s[t] * rb[t, sl]
            return 0

        lax.fori_loop(0, H // _L, body, 0)
        if c + 2 < 4:
            g0[c + 2] = pltpu.async_copy(
                ys_hbm.at[idx0[c + 2]], r0[c % 2], sg0[c % 2])
            g1[c + 2] = pltpu.async_copy(
                ys_hbm.at[idx1[c + 2]], r1[c % 2], sg1[c % 2])
        puts[c] = pltpu.async_copy(
            ov, out_hbm.at[pl.ds(tbase + c * 16, 16)], semo[c % 2])
    puts[2].wait()
    puts[3].wait()


def kernel(x, router_logits, w_gate, w_up, w_down):
    T, H = x.shape
    E, _, F = w_gate.shape

    pos0, pos1, w0r, w1r, meta = pl.pallas_call(
        _routing_body,
        out_shape=(
            jax.ShapeDtypeStruct((T,), jnp.int32),
            jax.ShapeDtypeStruct((T,), jnp.int32),
            jax.ShapeDtypeStruct((T, _L), jnp.float32),
            jax.ShapeDtypeStruct((T, _L), jnp.float32),
            jax.ShapeDtypeStruct((128, 1), jnp.int32),
        ),
    )(router_logits)

    p0f = pos0
    p1f = pos1

    mesh = plsc.VectorSubcoreMesh(core_axis_name="c", subcore_axis_name="s")

    dispatch = functools.partial(
        pl.kernel,
        out_type=jax.ShapeDtypeStruct((_SPAD, H), jnp.float32),
        mesh=mesh,
        scratch_types=(
            [pltpu.VMEM((32,), jnp.int32)] * 4
            + [pltpu.VMEM((32, H), jnp.float32)] * 2
            + [pltpu.SemaphoreType.DMA] * 6
        ),
    )(_dispatch_body)
    xs = dispatch(p0f, p1f, x)

    ys = pl.pallas_call(
        _gmm_body,
        grid_spec=pltpu.PrefetchScalarGridSpec(
            num_scalar_prefetch=1,
            grid=(_NB,),
            in_specs=[
                pl.BlockSpec((_BM, H), lambda i, be: (i, 0)),
                pl.BlockSpec((1, H, F), lambda i, be: (be[i, 0], 0, 0)),
                pl.BlockSpec((1, H, F), lambda i, be: (be[i, 0], 0, 0)),
                pl.BlockSpec((1, F, H), lambda i, be: (be[i, 0], 0, 0)),
            ],
            out_specs=pl.BlockSpec((_BM, H), lambda i, be: (i, 0)),
        ),
        out_shape=jax.ShapeDtypeStruct((_SPAD, H), jnp.float32),
    )(meta, xs, w_gate, w_up, w_down)

    combine = functools.partial(
        pl.kernel,
        out_type=jax.ShapeDtypeStruct((T, H), jnp.float32),
        mesh=mesh,
        scratch_types=(
            [pltpu.VMEM((16,), jnp.int32)] * 8
            + [pltpu.VMEM((64, _L), jnp.float32)] * 2
            + [pltpu.VMEM((16, H), jnp.float32)] * 4
            + [pltpu.VMEM((16, H), jnp.float32)] * 2
            + [pltpu.SemaphoreType.DMA] * 6
        ),
    )(_combine_body)
    out = combine(p0f, p1f, w0r, w1r, ys)
    return out
